# baseline (speedup 1.0000x reference)
import functools

import jax
import jax.numpy as jnp
from jax import lax
from jax.experimental import pallas as pl
from jax.experimental.pallas import tpu as pltpu
from jax.experimental.pallas import tpu_sc as plsc

D = 128
N_TASK = 128
N_DATA = 10000
NFLAT = N_DATA * N_TASK

NC = 2
NS = 16
NW = NC * NS
CW = 80
WAVE = 25

SLICE = NFLAT // NS
ZCH = 4000


def _make_build_b(k_chunks: int):
  _mesh = plsc.VectorSubcoreMesh(
      core_axis_name="c", subcore_axis_name="s", num_cores=NC, num_subcores=NS)

  @functools.partial(
      pl.kernel,
      out_type=jax.ShapeDtypeStruct((NC * NFLAT,), jnp.float32),
      mesh=_mesh,
      scratch_types=[
          pltpu.VMEM((k_chunks * CW,), jnp.int32),
          pltpu.VMEM((k_chunks * CW,), jnp.float32),
          pltpu.VMEM((ZCH,), jnp.float32),
          pltpu.VMEM_SHARED((NFLAT,), jnp.float32),
          pltpu.SemaphoreType.DMA,
          pltpu.SemaphoreType.DMA,
          pltpu.SemaphoreType.DMA,
      ],
  )
  def build_b(idx_hbm, val_hbm, out_hbm,
              idx_v, val_v, zero_v, acc, sem_in, sem_z, sem_s):
    c = lax.axis_index("c")
    s = lax.axis_index("s")
    wid = c * NS + s

    pte = k_chunks * CW
    d_idx = pltpu.async_copy(idx_hbm.at[pl.ds(wid * pte, pte)], idx_v, sem_in)
    d_val = pltpu.async_copy(val_hbm.at[pl.ds(wid * pte, pte)], val_v, sem_in)

    def zfill(i, carry):
      zero_v[pl.ds(i * 16, 16)] = jnp.zeros((16,), jnp.float32)
      return carry
    lax.fori_loop(0, ZCH // 16, zfill, 0)

    base = s * SLICE
    zdescs = [
        pltpu.async_copy(zero_v, acc.at[pl.ds(base + z * ZCH, ZCH)], sem_z)
        for z in range(SLICE // ZCH)
    ]

    d_idx.wait()
    d_val.wait()
    for dz in zdescs:
      dz.wait()
    plsc.subcore_barrier()

    def wave(w, carry):
      j0 = w * WAVE
      descs = [
          pltpu.async_copy(val_v.at[pl.ds((j0 + i) * CW, CW)],
                           acc.at[idx_v.at[pl.ds((j0 + i) * CW, CW)]],
                           sem_s, add=True)
          for i in range(WAVE)
      ]
      for dsc in descs:
        dsc.wait()
      return carry
    lax.fori_loop(0, k_chunks // WAVE, wave, 0)

    plsc.subcore_barrier()

    pltpu.sync_copy(acc.at[pl.ds(base, SLICE)],
                    out_hbm.at[pl.ds(c * NFLAT + base, SLICE)])

  return build_b


TR = 1000
NT = N_DATA // TR


def _dense_body(x_ref, bp_ref, te_ref, wm0_ref, ws0_ref,
                b0_ref, wm1_ref, ws1_ref, b1_ref, pred_ref, de_ref, to_ref,
                te_s, m0_s, bsum_s, hd0_s, aggt0_s, aggt1_s, ht0_s, ht1m_s,
                ht1_s):
  f32 = jnp.float32
  p = pl.program_id(0)
  t = pl.program_id(1)

  def mm(a, b):
    return lax.dot_general(a, b, (((1,), (0,)), ((), ())),
                           preferred_element_type=f32)

  def mmT(a, b):
    return lax.dot_general(a, b, (((0,), (0,)), ((), ())),
                           preferred_element_type=f32)

  @pl.when((p == 0) & (t == 0))
  def _init():
    te = te_ref[...]
    nrm = jnp.sqrt(jnp.sum(te * te, axis=1, keepdims=True))
    te_n = te / jnp.maximum(nrm, 1e-12)
    te_s[...] = te_n
    m0_s[...] = mm(te_n, wm0_ref[...])
    aggt0_s[...] = jnp.zeros((N_TASK, D), f32)
    aggt1_s[...] = jnp.zeros((N_TASK, D), f32)

  @pl.when(p == 0)
  def _phase0():
    bt = bp_ref[0] + bp_ref[1]
    rows = pl.ds(t * TR, TR)
    bsum_s[rows, :] = bt
    x_t = x_ref[...]
    hd0_t = jnp.maximum(
        mm(bt, m0_s[...]) + mm(x_t, ws0_ref[...]) + b0_ref[...], 0.0)
    hd0_s[rows, :] = hd0_t
    aggt0_s[...] += mmT(bt, x_t)
    aggt1_s[...] += mmT(bt, hd0_t)

  @pl.when((p == 1) & (t == 0))
  def _task_side():
    ht0 = jnp.maximum(
        mm(aggt0_s[...], wm0_ref[...]) + mm(te_s[...], ws0_ref[...])
        + b0_ref[...], 0.0)
    ht1 = (mm(aggt1_s[...], wm1_ref[...]) + mm(ht0, ws1_ref[...])
           + b1_ref[...])
    ht0_s[...] = ht0
    ht1m_s[...] = mm(ht0, wm1_ref[...])
    ht1_s[...] = ht1
    to_ref[...] = ht1

  @pl.when(p == 1)
  def _phase1():
    rows = pl.ds(t * TR, TR)
    bt = bsum_s[rows, :]
    hd1_t = (mm(bt, ht1m_s[...])
             + mm(hd0_s[rows, :], ws1_ref[...]) + b1_ref[...])
    de_ref[...] = hd1_t
    pred_ref[...] = lax.dot_general(hd1_t, ht1_s[...],
                                    (((1,), (1,)), ((), ())),
                                    preferred_element_type=f32)


def _dense_call(x, bpair, te, wm0, ws0, b0, wm1, ws1, b1):
  f32 = jnp.float32
  last = NT - 1

  def row_map(p, t):
    return (jnp.where(p == 0, t, last), 0)

  def bp_map(p, t):
    return (0, jnp.where(p == 0, t, last), 0)

  def out_map(p, t):
    return (jnp.where(p == 0, 0, t), 0)

  const2 = lambda p, t: (0, 0)
  return pl.pallas_call(
      _dense_body,
      grid=(2, NT),
      in_specs=[
          pl.BlockSpec((TR, D), row_map),
          pl.BlockSpec((NC, TR, N_TASK), bp_map),
          pl.BlockSpec((N_TASK, D), const2),
          pl.BlockSpec((D, D), const2),
          pl.BlockSpec((D, D), const2),
          pl.BlockSpec((1, D), const2),
          pl.BlockSpec((D, D), const2),
          pl.BlockSpec((D, D), const2),
          pl.BlockSpec((1, D), const2),
      ],
      out_specs=(
          pl.BlockSpec((TR, N_TASK), out_map),
          pl.BlockSpec((TR, D), out_map),
          pl.BlockSpec((N_TASK, D), const2),
      ),
      out_shape=(
          jax.ShapeDtypeStruct((N_DATA, N_TASK), f32),
          jax.ShapeDtypeStruct((N_DATA, D), f32),
          jax.ShapeDtypeStruct((N_TASK, D), f32),
      ),
      scratch_shapes=[
          pltpu.VMEM((N_TASK, D), f32),
          pltpu.VMEM((D, D), f32),
          pltpu.VMEM((N_DATA, N_TASK), f32),
          pltpu.VMEM((N_DATA, D), f32),
          pltpu.VMEM((N_TASK, D), f32),
          pltpu.VMEM((N_TASK, D), f32),
          pltpu.VMEM((N_TASK, D), f32),
          pltpu.VMEM((D, D), f32),
          pltpu.VMEM((N_TASK, D), f32),
      ],
      compiler_params=pltpu.CompilerParams(
          dimension_semantics=("arbitrary", "arbitrary")),
  )(x, bpair, te, wm0, ws0, b0, wm1, ws1, b1)


def kernel(graph_feature, graph_targets_id, graph_targets_id_batch,
           graph_targets_value, task_emb, W_msg0, W_self0, b0,
           W_msg1, W_self1, b1):
  e = graph_targets_id.shape[0]
  per = NW * CW
  assert e % per == 0, "edge count must tile over 32 subcores"
  k_chunks = e // per

  tid = graph_targets_id.astype(jnp.int32)
  tb = graph_targets_id_batch.astype(jnp.int32)
  flat = tb * N_TASK + tid

  bflat = _make_build_b(k_chunks)(flat, graph_targets_value)
  bpair = bflat.reshape(NC, N_DATA, N_TASK)

  pred, de, to = _dense_call(
      graph_feature, bpair, task_emb, W_msg0, W_self0,
      b0.reshape(1, D), W_msg1, W_self1, b1.reshape(1, D))
  return (pred, de, to)


# speedup vs baseline: 1.1087x; 1.1087x over previous
import functools

import jax
import jax.numpy as jnp
from jax import lax
from jax.experimental import pallas as pl
from jax.experimental.pallas import tpu as pltpu
from jax.experimental.pallas import tpu_sc as plsc

D = 128
N_TASK = 128
N_DATA = 10000
NFLAT = N_DATA * N_TASK

NC = 2
NS = 16
NW = NC * NS
CW = 80
WAVE = 25

SLICE = NFLAT // NS
ZCH = 4000


def _make_build_b(k_chunks: int):
  _mesh = plsc.VectorSubcoreMesh(
      core_axis_name="c", subcore_axis_name="s", num_cores=NC, num_subcores=NS)

  @functools.partial(
      pl.kernel,
      out_type=jax.ShapeDtypeStruct((NC * NFLAT,), jnp.float32),
      mesh=_mesh,
      scratch_types=[
          pltpu.VMEM((k_chunks * CW,), jnp.int32),
          pltpu.VMEM((k_chunks * CW,), jnp.float32),
          pltpu.VMEM((ZCH,), jnp.float32),
          pltpu.VMEM_SHARED((NFLAT,), jnp.float32),
          pltpu.SemaphoreType.DMA,
          pltpu.SemaphoreType.DMA,
          pltpu.SemaphoreType.DMA,
      ],
  )
  def build_b(idx_hbm, val_hbm, out_hbm,
              idx_v, val_v, zero_v, acc, sem_in, sem_z, sem_s):
    c = lax.axis_index("c")
    s = lax.axis_index("s")
    wid = c * NS + s

    pte = k_chunks * CW
    d_idx = pltpu.async_copy(idx_hbm.at[pl.ds(wid * pte, pte)], idx_v, sem_in)
    d_val = pltpu.async_copy(val_hbm.at[pl.ds(wid * pte, pte)], val_v, sem_in)

    def zfill(i, carry):
      zero_v[pl.ds(i * 16, 16)] = jnp.zeros((16,), jnp.float32)
      return carry
    lax.fori_loop(0, ZCH // 16, zfill, 0)

    base = s * SLICE
    zdescs = [
        pltpu.async_copy(zero_v, acc.at[pl.ds(base + z * ZCH, ZCH)], sem_z)
        for z in range(SLICE // ZCH)
    ]

    d_idx.wait()
    d_val.wait()
    for dz in zdescs:
      dz.wait()
    plsc.subcore_barrier()

    def wave(w, carry):
      j0 = w * WAVE
      descs = [
          pltpu.async_copy(val_v.at[pl.ds((j0 + i) * CW, CW)],
                           acc.at[idx_v.at[pl.ds((j0 + i) * CW, CW)]],
                           sem_s, add=True)
          for i in range(WAVE)
      ]
      for dsc in descs:
        dsc.wait()
      return carry
    lax.fori_loop(0, k_chunks // WAVE, wave, 0)

    plsc.subcore_barrier()

    pltpu.sync_copy(acc.at[pl.ds(base, SLICE)],
                    out_hbm.at[pl.ds(c * NFLAT + base, SLICE)])

  return build_b


TR = 5000
NT = N_DATA // TR


def _dense_body(x_ref, bp_ref, te_ref, wm0_ref, ws0_ref,
                b0_ref, wm1_ref, ws1_ref, b1_ref, pred_ref, de_ref, to_ref,
                te_s, m0_s, bsum_s, hd0_s, aggt0_s, aggt1_s, ht0_s, ht1m_s,
                ht1_s):
  f32 = jnp.float32
  p = pl.program_id(0)
  t = pl.program_id(1)

  def mm(a, b):
    return lax.dot_general(a, b, (((1,), (0,)), ((), ())),
                           preferred_element_type=f32)

  def mmT(a, b):
    return lax.dot_general(a, b, (((0,), (0,)), ((), ())),
                           preferred_element_type=f32)

  @pl.when((p == 0) & (t == 0))
  def _init():
    te = te_ref[...]
    nrm = jnp.sqrt(jnp.sum(te * te, axis=1, keepdims=True))
    te_n = te / jnp.maximum(nrm, 1e-12)
    te_s[...] = te_n
    m0_s[...] = mm(te_n, wm0_ref[...])
    aggt0_s[...] = jnp.zeros((N_TASK, D), f32)
    aggt1_s[...] = jnp.zeros((N_TASK, D), f32)

  @pl.when(p == 0)
  def _phase0():
    bt = bp_ref[0] + bp_ref[1]
    rows = pl.ds(t * TR, TR)
    bsum_s[rows, :] = bt
    x_t = x_ref[...]
    hd0_t = jnp.maximum(
        mm(bt, m0_s[...]) + mm(x_t, ws0_ref[...]) + b0_ref[...], 0.0)
    hd0_s[rows, :] = hd0_t
    aggt0_s[...] += mmT(bt, x_t)
    aggt1_s[...] += mmT(bt, hd0_t)

  @pl.when((p == 1) & (t == 0))
  def _task_side():
    ht0 = jnp.maximum(
        mm(aggt0_s[...], wm0_ref[...]) + mm(te_s[...], ws0_ref[...])
        + b0_ref[...], 0.0)
    ht1 = (mm(aggt1_s[...], wm1_ref[...]) + mm(ht0, ws1_ref[...])
           + b1_ref[...])
    ht0_s[...] = ht0
    ht1m_s[...] = mm(ht0, wm1_ref[...])
    ht1_s[...] = ht1
    to_ref[...] = ht1

  @pl.when(p == 1)
  def _phase1():
    rows = pl.ds(t * TR, TR)
    bt = bsum_s[rows, :]
    hd1_t = (mm(bt, ht1m_s[...])
             + mm(hd0_s[rows, :], ws1_ref[...]) + b1_ref[...])
    de_ref[...] = hd1_t
    pred_ref[...] = lax.dot_general(hd1_t, ht1_s[...],
                                    (((1,), (1,)), ((), ())),
                                    preferred_element_type=f32)


def _dense_call(x, bpair, te, wm0, ws0, b0, wm1, ws1, b1):
  f32 = jnp.float32
  last = NT - 1

  def row_map(p, t):
    return (jnp.where(p == 0, t, last), 0)

  def bp_map(p, t):
    return (0, jnp.where(p == 0, t, last), 0)

  def out_map(p, t):
    return (jnp.where(p == 0, 0, t), 0)

  const2 = lambda p, t: (0, 0)
  return pl.pallas_call(
      _dense_body,
      grid=(2, NT),
      in_specs=[
          pl.BlockSpec((TR, D), row_map),
          pl.BlockSpec((NC, TR, N_TASK), bp_map),
          pl.BlockSpec((N_TASK, D), const2),
          pl.BlockSpec((D, D), const2),
          pl.BlockSpec((D, D), const2),
          pl.BlockSpec((1, D), const2),
          pl.BlockSpec((D, D), const2),
          pl.BlockSpec((D, D), const2),
          pl.BlockSpec((1, D), const2),
      ],
      out_specs=(
          pl.BlockSpec((TR, N_TASK), out_map),
          pl.BlockSpec((TR, D), out_map),
          pl.BlockSpec((N_TASK, D), const2),
      ),
      out_shape=(
          jax.ShapeDtypeStruct((N_DATA, N_TASK), f32),
          jax.ShapeDtypeStruct((N_DATA, D), f32),
          jax.ShapeDtypeStruct((N_TASK, D), f32),
      ),
      scratch_shapes=[
          pltpu.VMEM((N_TASK, D), f32),
          pltpu.VMEM((D, D), f32),
          pltpu.VMEM((N_DATA, N_TASK), f32),
          pltpu.VMEM((N_DATA, D), f32),
          pltpu.VMEM((N_TASK, D), f32),
          pltpu.VMEM((N_TASK, D), f32),
          pltpu.VMEM((N_TASK, D), f32),
          pltpu.VMEM((D, D), f32),
          pltpu.VMEM((N_TASK, D), f32),
      ],
      compiler_params=pltpu.CompilerParams(
          dimension_semantics=("arbitrary", "arbitrary")),
  )(x, bpair, te, wm0, ws0, b0, wm1, ws1, b1)


def kernel(graph_feature, graph_targets_id, graph_targets_id_batch,
           graph_targets_value, task_emb, W_msg0, W_self0, b0,
           W_msg1, W_self1, b1):
  e = graph_targets_id.shape[0]
  per = NW * CW
  assert e % per == 0, "edge count must tile over 32 subcores"
  k_chunks = e // per

  tid = graph_targets_id.astype(jnp.int32)
  tb = graph_targets_id_batch.astype(jnp.int32)
  flat = tb * N_TASK + tid

  bflat = _make_build_b(k_chunks)(flat, graph_targets_value)
  bpair = bflat.reshape(NC, N_DATA, N_TASK)

  pred, de, to = _dense_call(
      graph_feature, bpair, task_emb, W_msg0, W_self0,
      b0.reshape(1, D), W_msg1, W_self1, b1.reshape(1, D))
  return (pred, de, to)


# speedup vs baseline: 1.1149x; 1.0056x over previous
import functools

import jax
import jax.numpy as jnp
from jax import lax
from jax.experimental import pallas as pl
from jax.experimental.pallas import tpu as pltpu
from jax.experimental.pallas import tpu_sc as plsc

D = 128
N_TASK = 128
N_DATA = 10000
NFLAT = N_DATA * N_TASK

NC = 2
NS = 16
NW = NC * NS
CW = 80
WAVE = 25

SLICE = NFLAT // NS
ZCH = 4000


def _make_build_b(k_chunks: int):
  _mesh = plsc.VectorSubcoreMesh(
      core_axis_name="c", subcore_axis_name="s", num_cores=NC, num_subcores=NS)

  @functools.partial(
      pl.kernel,
      out_type=jax.ShapeDtypeStruct((NC * NFLAT,), jnp.float32),
      mesh=_mesh,
      scratch_types=[
          pltpu.VMEM((k_chunks * CW,), jnp.int32),
          pltpu.VMEM((k_chunks * CW,), jnp.float32),
          pltpu.VMEM((ZCH,), jnp.float32),
          pltpu.VMEM_SHARED((NFLAT,), jnp.float32),
          pltpu.SemaphoreType.DMA,
          pltpu.SemaphoreType.DMA,
          pltpu.SemaphoreType.DMA,
      ],
  )
  def build_b(idx_hbm, val_hbm, out_hbm,
              idx_v, val_v, zero_v, acc, sem_in, sem_z, sem_s):
    c = lax.axis_index("c")
    s = lax.axis_index("s")
    wid = c * NS + s

    pte = k_chunks * CW
    d_idx = pltpu.async_copy(idx_hbm.at[pl.ds(wid * pte, pte)], idx_v, sem_in)
    d_val = pltpu.async_copy(val_hbm.at[pl.ds(wid * pte, pte)], val_v, sem_in)

    def zfill(i, carry):
      zero_v[pl.ds(i * 16, 16)] = jnp.zeros((16,), jnp.float32)
      return carry
    lax.fori_loop(0, ZCH // 16, zfill, 0)

    base = s * SLICE
    zdescs = [
        pltpu.async_copy(zero_v, acc.at[pl.ds(base + z * ZCH, ZCH)], sem_z)
        for z in range(SLICE // ZCH)
    ]

    d_idx.wait()
    d_val.wait()
    for dz in zdescs:
      dz.wait()
    plsc.subcore_barrier()

    def wave(w, carry):
      j0 = w * WAVE
      descs = [
          pltpu.async_copy(val_v.at[pl.ds((j0 + i) * CW, CW)],
                           acc.at[idx_v.at[pl.ds((j0 + i) * CW, CW)]],
                           sem_s, add=True)
          for i in range(WAVE)
      ]
      for dsc in descs:
        dsc.wait()
      return carry
    lax.fori_loop(0, k_chunks // WAVE, wave, 0)

    plsc.subcore_barrier()

    pltpu.sync_copy(acc.at[pl.ds(base, SLICE)],
                    out_hbm.at[pl.ds(c * NFLAT + base, SLICE)])

  return build_b


TR = 2000
NT = N_DATA // TR


def _dense_body(x_ref, bp_ref, te_ref, wm0_ref, ws0_ref,
                b0_ref, wm1_ref, ws1_ref, b1_ref, pred_ref, de_ref, to_ref,
                te_s, m0_s, bsum_s, hd0_s, aggt0_s, aggt1_s, ht0_s, ht1m_s,
                ht1_s):
  f32 = jnp.float32
  p = pl.program_id(0)
  t = pl.program_id(1)

  def mm(a, b):
    return lax.dot_general(a, b, (((1,), (0,)), ((), ())),
                           preferred_element_type=f32)

  def mmT(a, b):
    return lax.dot_general(a, b, (((0,), (0,)), ((), ())),
                           preferred_element_type=f32)

  @pl.when((p == 0) & (t == 0))
  def _init():
    te = te_ref[...]
    nrm = jnp.sqrt(jnp.sum(te * te, axis=1, keepdims=True))
    te_n = te / jnp.maximum(nrm, 1e-12)
    te_s[...] = te_n
    m0_s[...] = mm(te_n, wm0_ref[...])
    aggt0_s[...] = jnp.zeros((N_TASK, D), f32)
    aggt1_s[...] = jnp.zeros((N_TASK, D), f32)

  @pl.when(p == 0)
  def _phase0():
    bt = bp_ref[0] + bp_ref[1]
    rows = pl.ds(t * TR, TR)
    bsum_s[rows, :] = bt
    x_t = x_ref[...]
    hd0_t = jnp.maximum(
        mm(bt, m0_s[...]) + mm(x_t, ws0_ref[...]) + b0_ref[...], 0.0)
    hd0_s[rows, :] = hd0_t
    aggt0_s[...] += mmT(bt, x_t)
    aggt1_s[...] += mmT(bt, hd0_t)

  @pl.when((p == 1) & (t == 0))
  def _task_side():
    ht0 = jnp.maximum(
        mm(aggt0_s[...], wm0_ref[...]) + mm(te_s[...], ws0_ref[...])
        + b0_ref[...], 0.0)
    ht1 = (mm(aggt1_s[...], wm1_ref[...]) + mm(ht0, ws1_ref[...])
           + b1_ref[...])
    ht0_s[...] = ht0
    ht1m_s[...] = mm(ht0, wm1_ref[...])
    ht1_s[...] = ht1
    to_ref[...] = ht1

  @pl.when(p == 1)
  def _phase1():
    rows = pl.ds(t * TR, TR)
    bt = bsum_s[rows, :]
    hd1_t = (mm(bt, ht1m_s[...])
             + mm(hd0_s[rows, :], ws1_ref[...]) + b1_ref[...])
    de_ref[...] = hd1_t
    pred_ref[...] = lax.dot_general(hd1_t, ht1_s[...],
                                    (((1,), (1,)), ((), ())),
                                    preferred_element_type=f32)


def _dense_call(x, bpair, te, wm0, ws0, b0, wm1, ws1, b1):
  f32 = jnp.float32
  last = NT - 1

  def row_map(p, t):
    return (jnp.where(p == 0, t, last), 0)

  def bp_map(p, t):
    return (0, jnp.where(p == 0, t, last), 0)

  def out_map(p, t):
    return (jnp.where(p == 0, 0, t), 0)

  const2 = lambda p, t: (0, 0)
  return pl.pallas_call(
      _dense_body,
      grid=(2, NT),
      in_specs=[
          pl.BlockSpec((TR, D), row_map),
          pl.BlockSpec((NC, TR, N_TASK), bp_map),
          pl.BlockSpec((N_TASK, D), const2),
          pl.BlockSpec((D, D), const2),
          pl.BlockSpec((D, D), const2),
          pl.BlockSpec((1, D), const2),
          pl.BlockSpec((D, D), const2),
          pl.BlockSpec((D, D), const2),
          pl.BlockSpec((1, D), const2),
      ],
      out_specs=(
          pl.BlockSpec((TR, N_TASK), out_map),
          pl.BlockSpec((TR, D), out_map),
          pl.BlockSpec((N_TASK, D), const2),
      ),
      out_shape=(
          jax.ShapeDtypeStruct((N_DATA, N_TASK), f32),
          jax.ShapeDtypeStruct((N_DATA, D), f32),
          jax.ShapeDtypeStruct((N_TASK, D), f32),
      ),
      scratch_shapes=[
          pltpu.VMEM((N_TASK, D), f32),
          pltpu.VMEM((D, D), f32),
          pltpu.VMEM((N_DATA, N_TASK), f32),
          pltpu.VMEM((N_DATA, D), f32),
          pltpu.VMEM((N_TASK, D), f32),
          pltpu.VMEM((N_TASK, D), f32),
          pltpu.VMEM((N_TASK, D), f32),
          pltpu.VMEM((D, D), f32),
          pltpu.VMEM((N_TASK, D), f32),
      ],
      compiler_params=pltpu.CompilerParams(
          dimension_semantics=("arbitrary", "arbitrary")),
  )(x, bpair, te, wm0, ws0, b0, wm1, ws1, b1)


def kernel(graph_feature, graph_targets_id, graph_targets_id_batch,
           graph_targets_value, task_emb, W_msg0, W_self0, b0,
           W_msg1, W_self1, b1):
  e = graph_targets_id.shape[0]
  per = NW * CW
  assert e % per == 0, "edge count must tile over 32 subcores"
  k_chunks = e // per

  tid = graph_targets_id.astype(jnp.int32)
  tb = graph_targets_id_batch.astype(jnp.int32)
  flat = tb * N_TASK + tid

  bflat = _make_build_b(k_chunks)(flat, graph_targets_value)
  bpair = bflat.reshape(NC, N_DATA, N_TASK)

  pred, de, to = _dense_call(
      graph_feature, bpair, task_emb, W_msg0, W_self0,
      b0.reshape(1, D), W_msg1, W_self1, b1.reshape(1, D))
  return (pred, de, to)
